# baseline (speedup 1.0000x reference)
import jax
import jax.numpy as jnp
from jax.experimental import pallas as pl
from jax.experimental.pallas import tpu as pltpu


def _rope_block(ids_ref, cos_ref, sin_ref, t_ref, out_ref):
    n_tok = ids_ref.shape[-1]
    n_rows, d = cos_ref.shape
    ids = ids_ref[0, 0, :]
    row_iota = jax.lax.broadcasted_iota(jnp.int32, (n_tok, n_rows), 1)
    onehot = (ids[:, None] == row_iota).astype(jnp.bfloat16)
    lane = jax.lax.broadcasted_iota(jnp.int32, (n_rows, d), 1)
    sin_tab = jnp.where(lane % 2 == 0, -sin_ref[...], sin_ref[...])
    cos_g = jnp.dot(onehot, cos_ref[...].astype(jnp.bfloat16),
                    preferred_element_type=jnp.float32)
    sin_g = jnp.dot(onehot, sin_tab.astype(jnp.bfloat16),
                    preferred_element_type=jnp.float32)
    tb = t_ref[0]
    even = jax.lax.broadcasted_iota(jnp.int32, tb.shape, 2) % 2 == 0
    swap = jnp.where(even, pltpu.roll(tb, d - 1, 2), pltpu.roll(tb, 1, 2))
    out_ref[0] = tb * cos_g[None] + swap * sin_g[None]


def kernel(t, rope_ids, freqs_cos, freqs_sin):
    b, h, n, d = t.shape
    r = freqs_cos.shape[0]
    ids3 = rope_ids.reshape(b, 1, n)
    return pl.pallas_call(
        _rope_block,
        grid=(b,),
        in_specs=[
            pl.BlockSpec((1, 1, n), lambda i: (i, 0, 0)),
            pl.BlockSpec((r, d), lambda i: (0, 0)),
            pl.BlockSpec((r, d), lambda i: (0, 0)),
            pl.BlockSpec((1, h, n, d), lambda i: (i, 0, 0, 0)),
        ],
        out_specs=pl.BlockSpec((1, h, n, d), lambda i: (i, 0, 0, 0)),
        out_shape=jax.ShapeDtypeStruct((b, h, n, d), t.dtype),
    )(ids3, freqs_cos, freqs_sin, t)


# speedup vs baseline: 1.2070x; 1.2070x over previous
import jax
import jax.numpy as jnp
from jax.experimental import pallas as pl
from jax.experimental.pallas import tpu as pltpu


def _rope_block(ids_ref, cos_ref, sin_ref, t_ref, out_ref):
    n_tok = ids_ref.shape[-1]
    n_rows, d = cos_ref.shape
    ids = ids_ref[0, 0, :]
    row_iota = jax.lax.broadcasted_iota(jnp.int32, (n_tok, n_rows), 1)
    onehot = (ids[:, None] == row_iota).astype(jnp.bfloat16)
    lane = jax.lax.broadcasted_iota(jnp.int32, (n_rows, d), 1)
    sin_tab = jnp.where(lane % 2 == 0, -sin_ref[...], sin_ref[...])
    cos_g = jnp.dot(onehot, cos_ref[...].astype(jnp.bfloat16),
                    preferred_element_type=jnp.float32)
    sin_g = jnp.dot(onehot, sin_tab.astype(jnp.bfloat16),
                    preferred_element_type=jnp.float32)
    tb = t_ref[0]
    rowm = jax.lax.broadcasted_iota(jnp.int32, (d, d), 0)
    colm = jax.lax.broadcasted_iota(jnp.int32, (d, d), 1)
    m = ((rowm ^ 1) == colm).astype(jnp.bfloat16)
    h = tb.shape[0]
    t2 = tb.reshape(h * n_tok, d).astype(jnp.bfloat16)
    swap = jnp.dot(t2, m, preferred_element_type=jnp.float32).reshape(h, n_tok, d)
    out_ref[0] = tb * cos_g[None] + swap * sin_g[None]


def kernel(t, rope_ids, freqs_cos, freqs_sin):
    b, h, n, d = t.shape
    r = freqs_cos.shape[0]
    ids3 = rope_ids.reshape(b, 1, n)
    return pl.pallas_call(
        _rope_block,
        grid=(b,),
        in_specs=[
            pl.BlockSpec((1, 1, n), lambda i: (i, 0, 0)),
            pl.BlockSpec((r, d), lambda i: (0, 0)),
            pl.BlockSpec((r, d), lambda i: (0, 0)),
            pl.BlockSpec((1, h, n, d), lambda i: (i, 0, 0, 0)),
        ],
        out_specs=pl.BlockSpec((1, h, n, d), lambda i: (i, 0, 0, 0)),
        out_shape=jax.ShapeDtypeStruct((b, h, n, d), t.dtype),
    )(ids3, freqs_cos, freqs_sin, t)


# speedup vs baseline: 1.2146x; 1.0063x over previous
import jax
import jax.numpy as jnp
from jax.experimental import pallas as pl
from jax.experimental.pallas import tpu as pltpu

_HC = 8
_NBUF = 8
_OBUF = 8


def _gather_tables(ids, cos_ref, sin_ref):
    n_tok = ids.shape[0]
    n_rows, d = cos_ref.shape
    row_iota = jax.lax.broadcasted_iota(jnp.int32, (n_tok, n_rows), 1)
    onehot = (ids[:, None] == row_iota).astype(jnp.bfloat16)
    lane = jax.lax.broadcasted_iota(jnp.int32, (n_rows, d), 1)
    sin_tab = jnp.where(lane % 2 == 0, -sin_ref[...], sin_ref[...])
    cos_g = jnp.dot(onehot, cos_ref[...].astype(jnp.bfloat16),
                    preferred_element_type=jnp.float32)
    sin_g = jnp.dot(onehot, sin_tab.astype(jnp.bfloat16),
                    preferred_element_type=jnp.float32)
    return cos_g, sin_g


def _rotate_combine(tb, cos_g, sin_g):
    h, n_tok, d = tb.shape
    rowm = jax.lax.broadcasted_iota(jnp.int32, (d, d), 0)
    colm = jax.lax.broadcasted_iota(jnp.int32, (d, d), 1)
    m = ((rowm ^ 1) == colm).astype(jnp.bfloat16)
    t2 = tb.reshape(h * n_tok, d).astype(jnp.bfloat16)
    swap = jnp.dot(t2, m, preferred_element_type=jnp.float32).reshape(h, n_tok, d)
    return tb * cos_g[None] + swap * sin_g[None]


def _rope_manual(ids_ref, cos_ref, sin_ref, t_hbm, out_hbm,
                 in_buf, out_buf, in_sems, out_sems):
    b_total, h, n_tok, d = t_hbm.shape
    cpb = h // _HC
    nchunks = b_total * cpb

    def in_dma(c, slot):
        b = c // cpb
        hc = c % cpb
        return pltpu.make_async_copy(
            t_hbm.at[b, pl.ds(hc * _HC, _HC)], in_buf.at[slot],
            in_sems.at[slot])

    def out_dma(c, slot):
        b = c // cpb
        hc = c % cpb
        return pltpu.make_async_copy(
            out_buf.at[slot], out_hbm.at[b, pl.ds(hc * _HC, _HC)],
            out_sems.at[slot])

    for c in range(_NBUF):
        in_dma(c, c).start()

    def body(c, _):
        slot = jax.lax.rem(c, _NBUF)
        oslot = jax.lax.rem(c, _OBUF)
        b = c // cpb
        in_dma(c, slot).wait()
        ids = ids_ref[b, 0, :]
        cos_g, sin_g = _gather_tables(ids, cos_ref, sin_ref)
        res = _rotate_combine(in_buf[slot], cos_g, sin_g)

        @pl.when(c >= _OBUF)
        def _():
            out_dma(c - _OBUF, oslot).wait()

        out_buf[oslot] = res
        out_dma(c, oslot).start()

        @pl.when(c + _NBUF < nchunks)
        def _():
            in_dma(c + _NBUF, slot).start()

        return _

    jax.lax.fori_loop(0, nchunks, body, None)
    for k in range(_OBUF):
        c = nchunks - _OBUF + k
        out_dma(c, c % _OBUF).wait()


def kernel(t, rope_ids, freqs_cos, freqs_sin):
    b, h, n, d = t.shape
    r = freqs_cos.shape[0]
    ids3 = rope_ids.reshape(b, 1, n)
    return pl.pallas_call(
        _rope_manual,
        in_specs=[
            pl.BlockSpec(memory_space=pltpu.MemorySpace.VMEM),
            pl.BlockSpec(memory_space=pltpu.MemorySpace.VMEM),
            pl.BlockSpec(memory_space=pltpu.MemorySpace.VMEM),
            pl.BlockSpec(memory_space=pltpu.MemorySpace.HBM),
        ],
        out_specs=pl.BlockSpec(memory_space=pltpu.MemorySpace.HBM),
        out_shape=jax.ShapeDtypeStruct((b, h, n, d), t.dtype),
        scratch_shapes=[
            pltpu.VMEM((_NBUF, _HC, n, d), jnp.float32),
            pltpu.VMEM((_OBUF, _HC, n, d), jnp.float32),
            pltpu.SemaphoreType.DMA((_NBUF,)),
            pltpu.SemaphoreType.DMA((_OBUF,)),
        ],
    )(ids3, freqs_cos, freqs_sin, t)
